# baseline (speedup 1.0000x reference)
import functools
import math

import jax
import jax.numpy as jnp
from jax.experimental import pallas as pl
from jax.experimental.pallas import tpu as pltpu

_SMOOTH = 0.1
_SEQ_LEN = 4096
_ROWS = 512


def _body(x_ref, t_ref, o_ref, *, fill, conf, t1):
    i = pl.program_id(0)
    r, c = x_ref.shape
    tgt = t_ref[0, 0, :].reshape(r, 1)
    col = jax.lax.broadcasted_iota(jnp.int32, (r, c), 1)
    w = jnp.where(col == tgt, jnp.float32(conf), jnp.float32(fill))
    part = jnp.sum(w * jnp.log(x_ref[...]))

    @pl.when(i == 0)
    def _():
        o_ref[0, 0] = jnp.float32(t1)

    o_ref[0, 0] = o_ref[0, 0] - part


def kernel(x, target, device):
    b, s, c = x.shape
    n = b * s
    fill = _SMOOTH / _SEQ_LEN
    conf = 1.0 - _SMOOTH
    t1 = n * ((c - 1) * fill * math.log(fill) + conf * math.log(conf))

    x2 = x.reshape(n, c)
    nblk = n // _ROWS
    t3 = target.reshape(nblk, 1, _ROWS).astype(jnp.int32)

    body = functools.partial(_body, fill=fill, conf=conf, t1=t1)
    out = pl.pallas_call(
        body,
        grid=(nblk,),
        in_specs=[
            pl.BlockSpec((_ROWS, c), lambda i: (i, 0)),
            pl.BlockSpec((1, 1, _ROWS), lambda i: (i, 0, 0)),
        ],
        out_specs=pl.BlockSpec((1, 1), lambda i: (0, 0),
                               memory_space=pltpu.SMEM),
        out_shape=jax.ShapeDtypeStruct((1, 1), jnp.float32),
    )(x2, t3)
    return out[0, 0]


# speedup vs baseline: 1.0617x; 1.0617x over previous
import functools
import math

import jax
import jax.numpy as jnp
from jax.experimental import pallas as pl
from jax.experimental.pallas import tpu as pltpu

_SMOOTH = 0.1
_SEQ_LEN = 4096
_ROWS = 1024


def _body(x_ref, t_ref, o_ref, *, fill, conf, t1):
    i = pl.program_id(0)
    r, c = x_ref.shape
    tgt = t_ref[0, 0, :].reshape(r, 1)
    col = jax.lax.broadcasted_iota(jnp.int32, (r, c), 1)
    w = jnp.where(col == tgt, jnp.float32(conf), jnp.float32(fill))
    part = jnp.sum(w * jnp.log(x_ref[...]))

    @pl.when(i == 0)
    def _():
        o_ref[0, 0] = jnp.float32(t1)

    o_ref[0, 0] = o_ref[0, 0] - part


def kernel(x, target, device):
    b, s, c = x.shape
    n = b * s
    fill = _SMOOTH / _SEQ_LEN
    conf = 1.0 - _SMOOTH
    t1 = n * ((c - 1) * fill * math.log(fill) + conf * math.log(conf))

    x2 = x.reshape(n, c)
    nblk = n // _ROWS
    t3 = target.reshape(nblk, 1, _ROWS).astype(jnp.int32)

    body = functools.partial(_body, fill=fill, conf=conf, t1=t1)
    out = pl.pallas_call(
        body,
        grid=(nblk,),
        in_specs=[
            pl.BlockSpec((_ROWS, c), lambda i: (i, 0)),
            pl.BlockSpec((1, 1, _ROWS), lambda i: (i, 0, 0)),
        ],
        out_specs=pl.BlockSpec((1, 1), lambda i: (0, 0),
                               memory_space=pltpu.SMEM),
        out_shape=jax.ShapeDtypeStruct((1, 1), jnp.float32),
    )(x2, t3)
    return out[0, 0]
